# baseline (speedup 1.0000x reference)
import functools

import jax
import jax.numpy as jnp
from jax import lax
from jax.experimental import pallas as pl
from jax.experimental.pallas import tpu as pltpu
from jax.experimental.pallas import tpu_sc as plsc

_L = 16
_BAND = 32


def _sc_body(x_hbm, t_hbm, o_hbm, buf, tbuf, tailbuf, ttail):
    batch = x_hbm.shape[0]
    dim = x_hbm.shape[2]
    nvec = dim // _L
    wid = lax.axis_index("s") * 2 + lax.axis_index("c")
    r0 = wid * _BAND

    pltpu.sync_copy(t_hbm.at[pl.ds(r0, _BAND)], tbuf)

    @pl.when(wid == 31)
    def _():
        pltpu.sync_copy(t_hbm.at[pl.ds(_BAND * 32, 1)], ttail)

    def per_batch(b, carry):
        pltpu.sync_copy(x_hbm.at[b, pl.ds(r0, _BAND)], buf)

        def per_row(r, c2):
            for c in range(nvec):
                sl = pl.ds(c * _L, _L)
                buf[r, sl] = buf[r, sl] + tbuf[r, sl]
            return c2

        lax.fori_loop(0, _BAND, per_row, 0)
        pltpu.sync_copy(buf, o_hbm.at[b, pl.ds(r0, _BAND)])

        @pl.when(wid == 31)
        def _():
            pltpu.sync_copy(x_hbm.at[b, pl.ds(_BAND * 32, 1)], tailbuf)
            for c in range(nvec):
                sl = pl.ds(c * _L, _L)
                tailbuf[0, sl] = tailbuf[0, sl] + ttail[0, sl]
            pltpu.sync_copy(tailbuf, o_hbm.at[b, pl.ds(_BAND * 32, 1)])

        return carry

    lax.fori_loop(0, batch, per_batch, 0)


def kernel(inputs, pos_table):
    batch, positions, dim = inputs.shape
    mesh = plsc.VectorSubcoreMesh(core_axis_name="c", subcore_axis_name="s")
    sc_fn = functools.partial(
        pl.kernel,
        mesh=mesh,
        out_type=jax.ShapeDtypeStruct(inputs.shape, inputs.dtype),
        scratch_types=[
            pltpu.VMEM((_BAND, dim), inputs.dtype),
            pltpu.VMEM((_BAND, dim), inputs.dtype),
            pltpu.VMEM((1, dim), inputs.dtype),
            pltpu.VMEM((1, dim), inputs.dtype),
        ],
    )(_sc_body)
    return sc_fn(inputs, pos_table)


# speedup vs baseline: 1.4025x; 1.4025x over previous
import functools

import jax
import jax.numpy as jnp
from jax import lax
from jax.experimental import pallas as pl
from jax.experimental.pallas import tpu as pltpu
from jax.experimental.pallas import tpu_sc as plsc

_L = 16
_BAND = 32
_TAIL = _BAND * 32


def _sc_body(x_hbm, t_hbm, o_hbm,
             in0, in1, ou0, ou1, tbuf,
             tin0, tin1, tou0, tou1, ttail,
             si0, si1, so0, so1, tsi0, tsi1, tso0, tso1):
    batch = x_hbm.shape[0]
    dim = x_hbm.shape[2]
    nvec = dim // _L
    wid = lax.axis_index("s") * 2 + lax.axis_index("c")
    rows = pl.ds(wid * _BAND, _BAND)
    trow = pl.ds(_TAIL, 1)
    is_tail_tile = wid == 31

    in_bufs, out_bufs = (in0, in1), (ou0, ou1)
    in_sems, out_sems = (si0, si1), (so0, so1)
    tin_bufs, tout_bufs = (tin0, tin1), (tou0, tou1)
    tin_sems, tout_sems = (tsi0, tsi1), (tso0, tso1)

    pltpu.sync_copy(t_hbm.at[rows], tbuf)

    @pl.when(is_tail_tile)
    def _():
        pltpu.sync_copy(t_hbm.at[trow], ttail)

    def in_copy(b, j):
        return pltpu.make_async_copy(x_hbm.at[b, rows], in_bufs[j], in_sems[j])

    def out_copy(b, j):
        return pltpu.make_async_copy(out_bufs[j], o_hbm.at[b, rows], out_sems[j])

    def tin_copy(b, j):
        return pltpu.make_async_copy(x_hbm.at[b, trow], tin_bufs[j], tin_sems[j])

    def tout_copy(b, j):
        return pltpu.make_async_copy(tout_bufs[j], o_hbm.at[b, trow], tout_sems[j])

    in_copy(0, 0).start()
    in_copy(1, 1).start()

    @pl.when(is_tail_tile)
    def _():
        tin_copy(0, 0).start()
        tin_copy(1, 1).start()

    def round_fn(g, carry):
        for j in range(2):
            b = g * 2 + j
            in_copy(b, j).wait()

            @pl.when(b >= 2)
            def _():
                out_copy(b - 2, j).wait()

            def per_row(r, c2):
                for c in range(nvec):
                    sl = pl.ds(c * _L, _L)
                    out_bufs[j][r, sl] = in_bufs[j][r, sl] + tbuf[r, sl]
                return c2

            lax.fori_loop(0, _BAND, per_row, 0)
            out_copy(b, j).start()

            @pl.when(b + 2 < batch)
            def _():
                in_copy(b + 2, j).start()

            @pl.when(is_tail_tile)
            def _():
                tin_copy(b, j).wait()

                @pl.when(b >= 2)
                def _():
                    tout_copy(b - 2, j).wait()

                for c in range(nvec):
                    sl = pl.ds(c * _L, _L)
                    tout_bufs[j][0, sl] = tin_bufs[j][0, sl] + ttail[0, sl]
                tout_copy(b, j).start()

                @pl.when(b + 2 < batch)
                def _():
                    tin_copy(b + 2, j).start()

        return carry

    lax.fori_loop(0, batch // 2, round_fn, 0)
    out_copy(batch - 2, 0).wait()
    out_copy(batch - 1, 1).wait()

    @pl.when(is_tail_tile)
    def _():
        tout_copy(batch - 2, 0).wait()
        tout_copy(batch - 1, 1).wait()


def kernel(inputs, pos_table):
    batch, positions, dim = inputs.shape
    mesh = plsc.VectorSubcoreMesh(core_axis_name="c", subcore_axis_name="s")
    band = pltpu.VMEM((_BAND, dim), inputs.dtype)
    row = pltpu.VMEM((1, dim), inputs.dtype)
    sem = pltpu.SemaphoreType.DMA
    sc_fn = functools.partial(
        pl.kernel,
        mesh=mesh,
        out_type=jax.ShapeDtypeStruct(inputs.shape, inputs.dtype),
        scratch_types=[band, band, band, band, band,
                       row, row, row, row, row,
                       sem, sem, sem, sem, sem, sem, sem, sem],
    )(_sc_body)
    return sc_fn(inputs, pos_table)


# speedup vs baseline: 1.5087x; 1.0757x over previous
import jax
import jax.numpy as jnp
from jax.experimental import pallas as pl
from jax.experimental.pallas import tpu as pltpu

_NIN = 6
_NOUT = 6


def _pipeline_kernel(x_hbm, t_hbm, o_hbm, xbuf, obuf, tbuf, in_sem, out_sem, t_sem):
    batch = x_hbm.shape[0]

    tcopy = pltpu.make_async_copy(t_hbm, tbuf, t_sem)
    tcopy.start()

    def in_copy(b):
        slot = jax.lax.rem(b, _NIN)
        return pltpu.make_async_copy(x_hbm.at[b], xbuf.at[slot], in_sem.at[slot])

    def out_copy(b):
        slot = jax.lax.rem(b, _NOUT)
        return pltpu.make_async_copy(obuf.at[slot], o_hbm.at[b], out_sem.at[slot])

    for i in range(_NIN):
        in_copy(i).start(priority=0)

    tcopy.wait()

    def step(b, carry):
        si = jax.lax.rem(b, _NIN)
        so = jax.lax.rem(b, _NOUT)
        in_copy(b).wait()

        @pl.when(b >= _NOUT)
        def _():
            out_copy(b - _NOUT).wait()

        obuf[so] = xbuf[si] + tbuf[...]
        out_copy(b).start(priority=1)

        @pl.when(b + _NIN < batch)
        def _():
            in_copy(b + _NIN).start()

        return carry

    jax.lax.fori_loop(0, batch, step, 0)

    for i in range(_NOUT):
        out_copy(batch - _NOUT + i).wait()


def kernel(inputs, pos_table):
    batch, positions, dim = inputs.shape
    return pl.pallas_call(
        _pipeline_kernel,
        in_specs=[
            pl.BlockSpec(memory_space=pltpu.HBM),
            pl.BlockSpec(memory_space=pltpu.HBM),
        ],
        out_specs=pl.BlockSpec(memory_space=pltpu.HBM),
        out_shape=jax.ShapeDtypeStruct(inputs.shape, inputs.dtype),
        scratch_shapes=[
            pltpu.VMEM((_NIN, positions, dim), inputs.dtype),
            pltpu.VMEM((_NOUT, positions, dim), inputs.dtype),
            pltpu.VMEM((positions, dim), pos_table.dtype),
            pltpu.SemaphoreType.DMA((_NIN,)),
            pltpu.SemaphoreType.DMA((_NOUT,)),
            pltpu.SemaphoreType.DMA,
        ],
    )(inputs, pos_table)
